# Initial kernel scaffold; baseline (speedup 1.0000x reference)
import jax
import jax.numpy as jnp
from jax.experimental import pallas as pl


def kernel(pred, target):
    raise NotImplementedError("write your pallas kernel here")


# speedup vs baseline: 1.3439x; 1.3439x over previous
import functools

import jax
import jax.numpy as jnp
from jax import lax
from jax.experimental import pallas as pl
from jax.experimental.pallas import tpu as pltpu
from jax.experimental.pallas import tpu_sc as plsc

N_TOTAL = 8_000_000
NBINS = 10
NROWS = 2 * NBINS
LANES = 16
NWORKERS = 32
PER_W = N_TOTAL // NWORKERS
CHUNK = 10_000
NCHUNK = PER_W // CHUNK


def _sc_pass(pred, target):
    mesh = plsc.VectorSubcoreMesh(core_axis_name="c", subcore_axis_name="s")

    @functools.partial(
        pl.kernel,
        out_type=jax.ShapeDtypeStruct((NWORKERS, NROWS, LANES), jnp.float32),
        mesh=mesh,
        scratch_types=[
            pltpu.VMEM((CHUNK,), jnp.float32),
            pltpu.VMEM((CHUNK,), jnp.float32),
            pltpu.VMEM((CHUNK,), jnp.float32),
            pltpu.VMEM((CHUNK,), jnp.float32),
            pltpu.VMEM((NROWS, LANES), jnp.float32),
            pltpu.SemaphoreType.DMA,
            pltpu.SemaphoreType.DMA,
            pltpu.SemaphoreType.DMA,
            pltpu.SemaphoreType.DMA,
        ],
        compiler_params=pltpu.CompilerParams(needs_layout_passes=False),
    )
    def sc_kernel(pred_hbm, target_hbm, out_hbm, pbuf0, pbuf1, tbuf0, tbuf1, acc, sp0, sp1, st0, st1):
        pbufs = [pbuf0, pbuf1]
        tbufs = [tbuf0, tbuf1]
        wid = lax.axis_index("s") * 2 + lax.axis_index("c")
        base = wid * PER_W
        for r in range(NROWS):
            acc[r, :] = jnp.zeros((LANES,), jnp.float32)
        lane = lax.iota(jnp.int32, LANES)
        ones = jnp.ones((LANES,), jnp.float32)
        psem = [sp0, sp1]
        tsem = [st0, st1]

        def start(c, slot):
            off = base + c * CHUNK
            pltpu.make_async_copy(pred_hbm.at[pl.ds(off, CHUNK)], pbufs[slot], psem[slot]).start()
            pltpu.make_async_copy(target_hbm.at[pl.ds(off, CHUNK)], tbufs[slot], tsem[slot]).start()

        def wait(slot):
            pltpu.make_async_copy(pred_hbm.at[pl.ds(base, CHUNK)], pbufs[slot], psem[slot]).wait()
            pltpu.make_async_copy(target_hbm.at[pl.ds(base, CHUNK)], tbufs[slot], tsem[slot]).wait()

        start(0, 0)
        for c in range(NCHUNK):
            slot = c & 1
            if c + 1 < NCHUNK:
                start(c + 1, slot ^ 1)
            wait(slot)

            def body(j, carry):
                off = pl.multiple_of(j * LANES, LANES)
                p = pbufs[slot][pl.ds(off, LANES)]
                t = tbufs[slot][pl.ds(off, LANES)]
                d = p - t
                ad = jnp.abs(d)
                diff = jnp.minimum(ad, jnp.float32(360.0) - ad)
                bf = diff * jnp.float32(10.0 / 180.0)
                b = jnp.minimum(bf.astype(jnp.int32), 9)
                loss = jnp.where(
                    ad < jnp.float32(1.0),
                    jnp.float32(0.5) * ad * ad,
                    ad - jnp.float32(0.5),
                )
                plsc.addupdate_scatter(acc, [b, lane], ones)
                plsc.addupdate_scatter(acc, [b + 10, lane], loss)
                return carry

            lax.fori_loop(0, CHUNK // LANES, body, 0)

        pltpu.sync_copy(acc, out_hbm.at[wid])

    return sc_kernel(pred, target)


def _combine(x):
    def ck(x_ref, o_ref):
        total = jnp.float32(N_TOTAL)
        v = x_ref[...]
        s = jnp.sum(v, axis=1, keepdims=True)
        counts = s[0:NBINS]
        sums = s[NBINS:NROWS]
        accm = jnp.where(counts > 0, jnp.float32(0.1) * counts, jnp.float32(0.0))
        n = jnp.sum((counts > 0).astype(jnp.float32))
        n_safe = jnp.maximum(n, jnp.float32(1.0))
        w = jnp.where(
            accm > 0,
            total / (n_safe * jnp.maximum(accm, jnp.float32(1e-12))),
            jnp.float32(0.0),
        )
        walpha = jnp.where(
            w > 0,
            jnp.exp(jnp.float32(0.75) * jnp.log(jnp.maximum(w, jnp.float32(1e-30)))),
            jnp.float32(0.0),
        )
        tot = jnp.sum(jnp.where(counts > 0, walpha * sums, jnp.float32(0.0)))
        tot = jnp.where(n > 0, tot, jnp.sum(sums))
        o_ref[...] = jnp.reshape(tot / total, (1, 1))

    return pl.pallas_call(ck, out_shape=jax.ShapeDtypeStruct((1, 1), jnp.float32))(x)


def kernel(pred, target):
    p = pred.reshape(-1)
    t = target.reshape(-1)
    parts = _sc_pass(p, t)
    x = parts.transpose(1, 0, 2).reshape(NROWS, NWORKERS * LANES)
    return _combine(x)[0, 0]


# speedup vs baseline: 1.7286x; 1.2863x over previous
import functools

import jax
import jax.numpy as jnp
from jax import lax
from jax.experimental import pallas as pl
from jax.experimental.pallas import tpu as pltpu
from jax.experimental.pallas import tpu_sc as plsc

N_TOTAL = 8_000_000
NBINS = 10
NROWS = 2 * NBINS
LANES = 16
NWORKERS = 32
PER_W = N_TOTAL // NWORKERS
CHUNK = 10_000
NCHUNK = PER_W // CHUNK
UNROLL = 5


def _sc_pass(pred, target):
    mesh = plsc.VectorSubcoreMesh(core_axis_name="c", subcore_axis_name="s")

    @functools.partial(
        pl.kernel,
        out_type=jax.ShapeDtypeStruct((NWORKERS, NROWS, LANES), jnp.float32),
        mesh=mesh,
        scratch_types=[
            pltpu.VMEM((CHUNK,), jnp.float32),
            pltpu.VMEM((CHUNK,), jnp.float32),
            pltpu.VMEM((CHUNK,), jnp.float32),
            pltpu.VMEM((CHUNK,), jnp.float32),
            pltpu.VMEM((NROWS, LANES), jnp.float32),
            pltpu.SemaphoreType.DMA,
            pltpu.SemaphoreType.DMA,
            pltpu.SemaphoreType.DMA,
            pltpu.SemaphoreType.DMA,
        ],
        compiler_params=pltpu.CompilerParams(needs_layout_passes=False),
    )
    def sc_kernel(pred_hbm, target_hbm, out_hbm, pbuf0, pbuf1, tbuf0, tbuf1, acc, sp0, sp1, st0, st1):
        pbufs = [pbuf0, pbuf1]
        tbufs = [tbuf0, tbuf1]
        wid = lax.axis_index("s") * 2 + lax.axis_index("c")
        base = wid * PER_W
        for r in range(NROWS):
            acc[r, :] = jnp.zeros((LANES,), jnp.float32)
        lane = lax.iota(jnp.int32, LANES)
        ones = jnp.ones((LANES,), jnp.float32)
        psem = [sp0, sp1]
        tsem = [st0, st1]

        def start(c, slot):
            off = base + c * CHUNK
            pltpu.make_async_copy(pred_hbm.at[pl.ds(off, CHUNK)], pbufs[slot], psem[slot]).start()
            pltpu.make_async_copy(target_hbm.at[pl.ds(off, CHUNK)], tbufs[slot], tsem[slot]).start()

        def wait(slot):
            pltpu.make_async_copy(pred_hbm.at[pl.ds(base, CHUNK)], pbufs[slot], psem[slot]).wait()
            pltpu.make_async_copy(target_hbm.at[pl.ds(base, CHUNK)], tbufs[slot], tsem[slot]).wait()

        start(0, 0)
        for c in range(NCHUNK):
            slot = c & 1
            if c + 1 < NCHUNK:
                start(c + 1, slot ^ 1)
            wait(slot)

            @plsc.parallel_loop(0, CHUNK, step=LANES, unroll=UNROLL)
            def body(off):
                off = pl.multiple_of(off, LANES)
                p = pbufs[slot][pl.ds(off, LANES)]
                t = tbufs[slot][pl.ds(off, LANES)]
                d = p - t
                ad = jnp.abs(d)
                diff = jnp.minimum(ad, jnp.float32(360.0) - ad)
                bf = diff * jnp.float32(10.0 / 180.0)
                b = jnp.minimum(bf.astype(jnp.int32), 9)
                loss = jnp.where(
                    ad < jnp.float32(1.0),
                    jnp.float32(0.5) * ad * ad,
                    ad - jnp.float32(0.5),
                )
                plsc.addupdate_scatter(acc, [b, lane], ones)
                plsc.addupdate_scatter(acc, [b + 10, lane], loss)

        pltpu.sync_copy(acc, out_hbm.at[wid])

    return sc_kernel(pred, target)


def _combine(x):
    def ck(x_ref, o_ref):
        total = jnp.float32(N_TOTAL)
        v = x_ref[...]
        s = jnp.sum(v, axis=1, keepdims=True)
        counts = s[0:NBINS]
        sums = s[NBINS:NROWS]
        accm = jnp.where(counts > 0, jnp.float32(0.1) * counts, jnp.float32(0.0))
        n = jnp.sum((counts > 0).astype(jnp.float32))
        n_safe = jnp.maximum(n, jnp.float32(1.0))
        w = jnp.where(
            accm > 0,
            total / (n_safe * jnp.maximum(accm, jnp.float32(1e-12))),
            jnp.float32(0.0),
        )
        walpha = jnp.where(
            w > 0,
            jnp.exp(jnp.float32(0.75) * jnp.log(jnp.maximum(w, jnp.float32(1e-30)))),
            jnp.float32(0.0),
        )
        tot = jnp.sum(jnp.where(counts > 0, walpha * sums, jnp.float32(0.0)))
        tot = jnp.where(n > 0, tot, jnp.sum(sums))
        o_ref[...] = jnp.reshape(tot / total, (1, 1))

    return pl.pallas_call(ck, out_shape=jax.ShapeDtypeStruct((1, 1), jnp.float32))(x)


def kernel(pred, target):
    p = pred.reshape(-1)
    t = target.reshape(-1)
    parts = _sc_pass(p, t)
    x = parts.transpose(1, 0, 2).reshape(NROWS, NWORKERS * LANES)
    return _combine(x)[0, 0]


# speedup vs baseline: 14.2432x; 8.2398x over previous
import functools

import jax
import jax.numpy as jnp
from jax import lax
from jax.experimental import pallas as pl
from jax.experimental.pallas import tpu as pltpu
from jax.experimental.pallas import tpu_sc as plsc

N_TOTAL = 8_000_000
NBINS = 10
NROWS = 2 * NBINS
LANES = 16
NWORKERS = 32
TILE = 128
PER_W = (N_TOTAL // TILE // NWORKERS) * TILE
CHUNK = 93 * TILE
NCHUNK = PER_W // CHUNK
UNROLL = 8
TAIL_OFF = NWORKERS * PER_W
TAIL = N_TOTAL - TAIL_OFF


def _sc_pass(pred, target):
    mesh = plsc.VectorSubcoreMesh(core_axis_name="c", subcore_axis_name="s")

    @functools.partial(
        pl.kernel,
        out_type=jax.ShapeDtypeStruct((NWORKERS, NROWS, LANES), jnp.float32),
        name="ghmr_sc_pass",
        mesh=mesh,
        scratch_types=[
            pltpu.VMEM((CHUNK,), jnp.float32),
            pltpu.VMEM((CHUNK,), jnp.float32),
            pltpu.VMEM((CHUNK,), jnp.float32),
            pltpu.VMEM((CHUNK,), jnp.float32),
            pltpu.VMEM((NROWS, LANES), jnp.float32),
            pltpu.SemaphoreType.DMA,
            pltpu.SemaphoreType.DMA,
            pltpu.SemaphoreType.DMA,
            pltpu.SemaphoreType.DMA,
        ],
        compiler_params=pltpu.CompilerParams(needs_layout_passes=False),
    )
    def sc_kernel(pred_hbm, target_hbm, out_hbm, pbuf0, pbuf1, tbuf0, tbuf1, acc, sp0, sp1, st0, st1):
        pbufs = [pbuf0, pbuf1]
        tbufs = [tbuf0, tbuf1]
        wid = lax.axis_index("s") * 2 + lax.axis_index("c")
        base = wid * PER_W
        for r in range(NROWS):
            acc[r, :] = jnp.zeros((LANES,), jnp.float32)
        lane = lax.iota(jnp.int32, LANES)
        ones = jnp.ones((LANES,), jnp.float32)
        psem = [sp0, sp1]
        tsem = [st0, st1]

        def start(c, slot):
            off = pl.multiple_of(base + c * CHUNK, TILE)
            pltpu.make_async_copy(pred_hbm.at[0, pl.ds(off, CHUNK)], pbufs[slot], psem[slot]).start()
            pltpu.make_async_copy(target_hbm.at[0, pl.ds(off, CHUNK)], tbufs[slot], tsem[slot]).start()

        def wait(slot):
            pltpu.make_async_copy(pred_hbm.at[0, pl.ds(base, CHUNK)], pbufs[slot], psem[slot]).wait()
            pltpu.make_async_copy(target_hbm.at[0, pl.ds(base, CHUNK)], tbufs[slot], tsem[slot]).wait()

        def vec_body(pbuf, tbuf, off):
            p = pbuf[pl.ds(off, LANES)]
            t = tbuf[pl.ds(off, LANES)]
            d = p - t
            ad = jnp.abs(d)
            diff = jnp.minimum(ad, jnp.float32(360.0) - ad)
            bf = diff * jnp.float32(10.0 / 180.0)
            b = jnp.minimum(bf.astype(jnp.int32), 9)
            loss = jnp.where(
                ad < jnp.float32(1.0),
                jnp.float32(0.5) * ad * ad,
                ad - jnp.float32(0.5),
            )
            plsc.addupdate_scatter(acc, [b, lane], ones)
            plsc.addupdate_scatter(acc, [b + 10, lane], loss)

        start(0, 0)
        for c in range(NCHUNK):
            slot = c & 1
            if c + 1 < NCHUNK:
                start(c + 1, slot ^ 1)
            wait(slot)

            @plsc.parallel_loop(0, CHUNK, step=LANES, unroll=UNROLL)
            def body(off):
                vec_body(pbufs[slot], tbufs[slot], pl.multiple_of(off, LANES))

        @pl.when(wid == 0)
        def _tail():
            pltpu.make_async_copy(
                pred_hbm.at[0, pl.ds(TAIL_OFF, TAIL)], pbufs[0].at[pl.ds(0, TAIL)], psem[0]
            ).start()
            pltpu.make_async_copy(
                target_hbm.at[0, pl.ds(TAIL_OFF, TAIL)], tbufs[0].at[pl.ds(0, TAIL)], tsem[0]
            ).start()
            pltpu.make_async_copy(
                pred_hbm.at[0, pl.ds(TAIL_OFF, TAIL)], pbufs[0].at[pl.ds(0, TAIL)], psem[0]
            ).wait()
            pltpu.make_async_copy(
                target_hbm.at[0, pl.ds(TAIL_OFF, TAIL)], tbufs[0].at[pl.ds(0, TAIL)], tsem[0]
            ).wait()

            @plsc.parallel_loop(0, TAIL, step=LANES, unroll=UNROLL)
            def tail_body(off):
                vec_body(pbufs[0], tbufs[0], pl.multiple_of(off, LANES))

        pltpu.sync_copy(acc, out_hbm.at[wid])

    return sc_kernel(pred, target)


def _combine(x):
    def ck(x_ref, o_ref):
        total = jnp.float32(N_TOTAL)
        v = x_ref[...]
        s = jnp.sum(v, axis=1, keepdims=True)
        counts = s[0:NBINS]
        sums = s[NBINS:NROWS]
        accm = jnp.where(counts > 0, jnp.float32(0.1) * counts, jnp.float32(0.0))
        n = jnp.sum((counts > 0).astype(jnp.float32))
        n_safe = jnp.maximum(n, jnp.float32(1.0))
        w = jnp.where(
            accm > 0,
            total / (n_safe * jnp.maximum(accm, jnp.float32(1e-12))),
            jnp.float32(0.0),
        )
        walpha = jnp.where(
            w > 0,
            jnp.exp(jnp.float32(0.75) * jnp.log(jnp.maximum(w, jnp.float32(1e-30)))),
            jnp.float32(0.0),
        )
        tot = jnp.sum(jnp.where(counts > 0, walpha * sums, jnp.float32(0.0)))
        tot = jnp.where(n > 0, tot, jnp.sum(sums))
        o_ref[...] = jnp.reshape(tot / total, (1, 1))

    return pl.pallas_call(ck, out_shape=jax.ShapeDtypeStruct((1, 1), jnp.float32))(x)


def kernel(pred, target):
    parts = _sc_pass(pred.T, target.T)
    x = parts.transpose(1, 0, 2).reshape(NROWS, NWORKERS * LANES)
    return _combine(x)[0, 0]


# speedup vs baseline: 14.4774x; 1.0164x over previous
import functools

import jax
import jax.numpy as jnp
from jax import lax
from jax.experimental import pallas as pl
from jax.experimental.pallas import tpu as pltpu
from jax.experimental.pallas import tpu_sc as plsc

N_TOTAL = 8_000_000
NBINS = 10
NROWS = 2 * NBINS
LANES = 16
NWORKERS = 32
TILE = 128
PER_W = (N_TOTAL // TILE // NWORKERS) * TILE
CHUNK = 217 * TILE
NCHUNK = PER_W // CHUNK
UNROLL = 8
TAIL_OFF = NWORKERS * PER_W
TAIL = N_TOTAL - TAIL_OFF


def _sc_pass(pred, target):
    mesh = plsc.VectorSubcoreMesh(core_axis_name="c", subcore_axis_name="s")

    @functools.partial(
        pl.kernel,
        out_type=jax.ShapeDtypeStruct((NWORKERS, NROWS, LANES), jnp.float32),
        name="ghmr_sc_pass",
        mesh=mesh,
        scratch_types=[
            pltpu.VMEM((CHUNK,), jnp.float32),
            pltpu.VMEM((CHUNK,), jnp.float32),
            pltpu.VMEM((CHUNK,), jnp.float32),
            pltpu.VMEM((CHUNK,), jnp.float32),
            pltpu.VMEM((NROWS, LANES), jnp.float32),
            pltpu.SemaphoreType.DMA,
            pltpu.SemaphoreType.DMA,
            pltpu.SemaphoreType.DMA,
            pltpu.SemaphoreType.DMA,
        ],
        compiler_params=pltpu.CompilerParams(needs_layout_passes=False),
    )
    def sc_kernel(pred_hbm, target_hbm, out_hbm, pbuf0, pbuf1, tbuf0, tbuf1, acc, sp0, sp1, st0, st1):
        pbufs = [pbuf0, pbuf1]
        tbufs = [tbuf0, tbuf1]
        wid = lax.axis_index("s") * 2 + lax.axis_index("c")
        base = wid * PER_W
        for r in range(NROWS):
            acc[r, :] = jnp.zeros((LANES,), jnp.float32)
        lane = lax.iota(jnp.int32, LANES)
        ones = jnp.ones((LANES,), jnp.float32)
        psem = [sp0, sp1]
        tsem = [st0, st1]

        def start(c, slot):
            off = pl.multiple_of(base + c * CHUNK, TILE)
            pltpu.make_async_copy(pred_hbm.at[0, pl.ds(off, CHUNK)], pbufs[slot], psem[slot]).start()
            pltpu.make_async_copy(target_hbm.at[0, pl.ds(off, CHUNK)], tbufs[slot], tsem[slot]).start()

        def wait(slot):
            pltpu.make_async_copy(pred_hbm.at[0, pl.ds(base, CHUNK)], pbufs[slot], psem[slot]).wait()
            pltpu.make_async_copy(target_hbm.at[0, pl.ds(base, CHUNK)], tbufs[slot], tsem[slot]).wait()

        def vec_body(pbuf, tbuf, off):
            p = pbuf[pl.ds(off, LANES)]
            t = tbuf[pl.ds(off, LANES)]
            d = p - t
            ad = jnp.abs(d)
            diff = jnp.minimum(ad, jnp.float32(360.0) - ad)
            bf = diff * jnp.float32(10.0 / 180.0)
            b = jnp.minimum(bf.astype(jnp.int32), 9)
            m = jnp.minimum(ad, jnp.float32(1.0))
            loss = jnp.float32(0.5) * m * m + (ad - m)
            plsc.addupdate_scatter(acc, [b, lane], ones)
            plsc.addupdate_scatter(acc, [b + 10, lane], loss)

        start(0, 0)
        for c in range(NCHUNK):
            slot = c & 1
            if c + 1 < NCHUNK:
                start(c + 1, slot ^ 1)
            wait(slot)

            @plsc.parallel_loop(0, CHUNK, step=LANES, unroll=UNROLL)
            def body(off):
                vec_body(pbufs[slot], tbufs[slot], pl.multiple_of(off, LANES))

        @pl.when(wid == 0)
        def _tail():
            pltpu.make_async_copy(
                pred_hbm.at[0, pl.ds(TAIL_OFF, TAIL)], pbufs[0].at[pl.ds(0, TAIL)], psem[0]
            ).start()
            pltpu.make_async_copy(
                target_hbm.at[0, pl.ds(TAIL_OFF, TAIL)], tbufs[0].at[pl.ds(0, TAIL)], tsem[0]
            ).start()
            pltpu.make_async_copy(
                pred_hbm.at[0, pl.ds(TAIL_OFF, TAIL)], pbufs[0].at[pl.ds(0, TAIL)], psem[0]
            ).wait()
            pltpu.make_async_copy(
                target_hbm.at[0, pl.ds(TAIL_OFF, TAIL)], tbufs[0].at[pl.ds(0, TAIL)], tsem[0]
            ).wait()

            @plsc.parallel_loop(0, TAIL, step=LANES, unroll=UNROLL)
            def tail_body(off):
                vec_body(pbufs[0], tbufs[0], pl.multiple_of(off, LANES))

        pltpu.sync_copy(acc, out_hbm.at[wid])

    return sc_kernel(pred, target)


def _combine(x):
    def ck(x_ref, o_ref):
        total = jnp.float32(N_TOTAL)
        v = x_ref[...]
        s = jnp.sum(v, axis=1, keepdims=True)
        counts = s[0:NBINS]
        sums = s[NBINS:NROWS]
        accm = jnp.where(counts > 0, jnp.float32(0.1) * counts, jnp.float32(0.0))
        n = jnp.sum((counts > 0).astype(jnp.float32))
        n_safe = jnp.maximum(n, jnp.float32(1.0))
        w = jnp.where(
            accm > 0,
            total / (n_safe * jnp.maximum(accm, jnp.float32(1e-12))),
            jnp.float32(0.0),
        )
        walpha = jnp.where(
            w > 0,
            jnp.exp(jnp.float32(0.75) * jnp.log(jnp.maximum(w, jnp.float32(1e-30)))),
            jnp.float32(0.0),
        )
        tot = jnp.sum(jnp.where(counts > 0, walpha * sums, jnp.float32(0.0)))
        tot = jnp.where(n > 0, tot, jnp.sum(sums))
        o_ref[...] = jnp.reshape(tot / total, (1, 1))

    return pl.pallas_call(ck, out_shape=jax.ShapeDtypeStruct((1, 1), jnp.float32))(x)


def kernel(pred, target):
    parts = _sc_pass(pred.T, target.T)
    x = parts.transpose(1, 0, 2).reshape(NROWS, NWORKERS * LANES)
    return _combine(x)[0, 0]
